# baseline (speedup 1.0000x reference)
import jax
import jax.numpy as jnp
from jax import lax
from jax.experimental import pallas as pl
from jax.experimental.pallas import tpu as pltpu
from jax.experimental.pallas import tpu_sc as plsc

_B = 4
_S = 8192
_D = 1024
_N = _B * _S

_INFO = plsc.get_sparse_core_info()
_NC = _INFO.num_cores
_NS = _INFO.num_subcores
_LANES = _INFO.num_lanes
_NW = _NC * _NS
_PER_W = _N // _NW
_CHUNK = 8
_NCHUNK = _PER_W // _CHUNK
_VECS = _D // _LANES
_NG = 8
_NX = 4


def _body(x_hbm, idx_hbm, table_hbm, out_hbm, idx_v, *bufs):
    rows = bufs[0:_NG]
    xb = bufs[_NG:_NG + _NX]
    gsem = bufs[_NG + _NX:2 * _NG + _NX]
    xsem = bufs[2 * _NG + _NX:2 * _NG + 2 * _NX]
    ssem = bufs[2 * _NG + 2 * _NX:2 * _NG + 3 * _NX]

    wid = lax.axis_index("s") * _NC + lax.axis_index("c")
    base = wid * _PER_W

    pltpu.sync_copy(idx_hbm.at[pl.ds(base, _PER_W)], idx_v)

    def issue_gather(k, b):
        pltpu.async_copy(
            table_hbm.at[idx_v.at[pl.ds(k * _CHUNK, _CHUNK)]], rows[b],
            gsem[b])

    def issue_xload(k, b):
        pltpu.async_copy(
            x_hbm.at[pl.ds(base + k * _CHUNK, _CHUNK), :], xb[b], xsem[b])

    def drain_store(k, b):
        pltpu.make_async_copy(
            xb[b], out_hbm.at[pl.ds(base + k * _CHUNK, _CHUNK), :],
            ssem[b]).wait()

    for kk in range(_NG - 1):
        issue_gather(kk, kk)
    for kk in range(2):
        issue_xload(kk, kk)

    @pl.loop(0, _NCHUNK, step=_NG)
    def _ring(g):
        for b in range(_NG):
            k = g + b
            bx = b % _NX

            @pl.when(k + _NG - 1 < _NCHUNK)
            def _prefetch_gather():
                issue_gather(k + _NG - 1, (b + _NG - 1) % _NG)

            @pl.when(k - 2 >= 0)
            def _drain():
                drain_store(k - 2, (bx + 2) % _NX)

            @pl.when(k + 2 < _NCHUNK)
            def _prefetch_xload():
                issue_xload(k + 2, (bx + 2) % _NX)

            pltpu.make_async_copy(
                table_hbm.at[idx_v.at[pl.ds(0, _CHUNK)]], rows[b],
                gsem[b]).wait()
            pltpu.make_async_copy(
                x_hbm.at[pl.ds(base, _CHUNK), :], xb[bx], xsem[bx]).wait()

            @pl.loop(0, _CHUNK)
            def _row(j):
                for l in range(_VECS):
                    sl = pl.ds(l * _LANES, _LANES)
                    xb[bx][j, sl] = xb[bx][j, sl] + rows[b][j, sl]

            pltpu.async_copy(
                xb[bx], out_hbm.at[pl.ds(base + k * _CHUNK, _CHUNK), :],
                ssem[bx])

    for kk in range(_NCHUNK - 2, _NCHUNK):
        drain_store(kk, kk % _NX)


@jax.jit
def _run(x2d, idx, table):
    mesh = plsc.VectorSubcoreMesh(core_axis_name="c", subcore_axis_name="s")
    return pl.kernel(
        _body,
        out_type=jax.ShapeDtypeStruct((_N, _D), jnp.float32),
        mesh=mesh,
        scratch_types=(
            [pltpu.VMEM((_PER_W,), jnp.int32)]
            + [pltpu.VMEM((_CHUNK, _D), jnp.float32)] * (_NG + _NX)
            + [pltpu.SemaphoreType.DMA] * (_NG + 2 * _NX)
        ),
    )(x2d, idx, table)


def kernel(x, positional_ids, table):
    x2d = x.reshape(_N, _D)
    idx = positional_ids.reshape(_N).astype(jnp.int32)
    out = _run(x2d, idx, table)
    return out.reshape(_B, _S, _D)


# speedup vs baseline: 1.0321x; 1.0321x over previous
import jax
import jax.numpy as jnp
from jax import lax
from jax.experimental import pallas as pl
from jax.experimental.pallas import tpu as pltpu
from jax.experimental.pallas import tpu_sc as plsc

_B = 4
_S = 8192
_D = 1024
_N = _B * _S

_INFO = plsc.get_sparse_core_info()
_NC = _INFO.num_cores
_NS = _INFO.num_subcores
_LANES = _INFO.num_lanes
_NW = _NC * _NS
_PER_W = _N // _NW
_CHUNK = 8
_NCHUNK = _PER_W // _CHUNK
_VECS = _D // _LANES
_NBUF = 4


def _body(x_hbm, idx_hbm, table_hbm, out_hbm, idx_v, *bufs):
    rows = bufs[0:_NBUF]
    xb = bufs[_NBUF:2 * _NBUF]
    gsem = bufs[2 * _NBUF:3 * _NBUF]
    xsem = bufs[3 * _NBUF:4 * _NBUF]
    ssem = bufs[4 * _NBUF:5 * _NBUF]

    wid = lax.axis_index("s") * _NC + lax.axis_index("c")
    base = wid * _PER_W

    pltpu.sync_copy(idx_hbm.at[pl.ds(base, _PER_W)], idx_v)

    def issue_gather(k, b):
        pltpu.async_copy(
            table_hbm.at[idx_v.at[pl.ds(k * _CHUNK, _CHUNK)]], rows[b],
            gsem[b])

    def issue_xload(k, b):
        pltpu.async_copy(
            x_hbm.at[pl.ds(base + k * _CHUNK, _CHUNK), :], xb[b], xsem[b])

    def drain_store(k, b):
        pltpu.make_async_copy(
            xb[b], out_hbm.at[pl.ds(base + k * _CHUNK, _CHUNK), :],
            ssem[b]).wait()

    for kk in range(_NBUF - 1):
        issue_gather(kk, kk)
    for kk in range(_NBUF - 2):
        issue_xload(kk, kk)

    @pl.loop(0, _NCHUNK, step=_NBUF)
    def _ring(g):
        for b in range(_NBUF):
            k = g + b

            @pl.when(k + _NBUF - 1 < _NCHUNK)
            def _prefetch_gather():
                issue_gather(k + _NBUF - 1, (b + _NBUF - 1) % _NBUF)

            @pl.when(k - 2 >= 0)
            def _drain():
                drain_store(k - 2, (b + 2) % _NBUF)

            @pl.when(k + 2 < _NCHUNK)
            def _prefetch_xload():
                issue_xload(k + 2, (b + 2) % _NBUF)

            pltpu.make_async_copy(
                table_hbm.at[idx_v.at[pl.ds(0, _CHUNK)]], rows[b],
                gsem[b]).wait()
            pltpu.make_async_copy(
                x_hbm.at[pl.ds(base, _CHUNK), :], xb[b], xsem[b]).wait()

            @pl.loop(0, _CHUNK)
            def _row(j):
                for l in range(_VECS):
                    sl = pl.ds(l * _LANES, _LANES)
                    xb[b][j, sl] = xb[b][j, sl] + rows[b][j, sl]

            pltpu.async_copy(
                xb[b], out_hbm.at[pl.ds(base + k * _CHUNK, _CHUNK), :],
                ssem[b])

    for kk in range(_NCHUNK - 2, _NCHUNK):
        drain_store(kk, kk % _NBUF)


@jax.jit
def _run(x2d, idx, table):
    mesh = plsc.VectorSubcoreMesh(core_axis_name="c", subcore_axis_name="s")
    return pl.kernel(
        _body,
        out_type=jax.ShapeDtypeStruct((_N, _D), jnp.float32),
        mesh=mesh,
        scratch_types=(
            [pltpu.VMEM((_PER_W,), jnp.int32)]
            + [pltpu.VMEM((_CHUNK, _D), jnp.float32)] * (2 * _NBUF)
            + [pltpu.SemaphoreType.DMA] * (3 * _NBUF)
        ),
    )(x2d, idx, table)


def kernel(x, positional_ids, table):
    x2d = x.reshape(_N, _D)
    idx = positional_ids.reshape(_N).astype(jnp.int32)
    out = _run(x2d, idx, table)
    return out.reshape(_B, _S, _D)
